# baseline (speedup 1.0000x reference)
import jax
import jax.numpy as jnp
from jax.experimental import pallas as pl
from jax.experimental.pallas import tpu as pltpu

_PACK = 8
_BM = 5000


def _mlp_body(x_ref, w1_ref, b1_ref, w2_ref, b2_ref, la_ref, lb_ref):
    x = x_ref[...]
    h = jnp.dot(x, w1_ref[...], preferred_element_type=jnp.float32)
    h = h + b1_ref[...]
    h = jnp.maximum(h, 0.01 * h)
    o = jnp.dot(h, w2_ref[...], preferred_element_type=jnp.float32)
    o = o + b2_ref[...]
    la_ref[...] = o[:, 0:8]
    lb_ref[...] = o[:, 8:16]


@jax.jit
def kernel(X, W1, b1, W2, b2):
    n, d = X.shape
    xr = X.reshape(n // _PACK, d * _PACK)

    eye = jnp.eye(_PACK, dtype=X.dtype)
    w1bd = jnp.kron(eye, W1)
    b1bd = jnp.tile(b1, _PACK).reshape(1, d * _PACK)
    w2a = jnp.kron(eye, W2[:, 0:1])
    w2b = jnp.kron(eye, W2[:, 1:2])
    w2bd = jnp.concatenate([w2a, w2b], axis=1)
    b2bd = jnp.concatenate(
        [jnp.full((_PACK,), b2[0], X.dtype), jnp.full((_PACK,), b2[1], X.dtype)]
    ).reshape(1, 2 * _PACK)

    nr = n // _PACK
    grid = (nr // _BM,)
    la, lb = pl.pallas_call(
        _mlp_body,
        grid=grid,
        in_specs=[
            pl.BlockSpec((_BM, d * _PACK), lambda i: (i, 0)),
            pl.BlockSpec((d * _PACK, d * _PACK), lambda i: (0, 0)),
            pl.BlockSpec((1, d * _PACK), lambda i: (0, 0)),
            pl.BlockSpec((d * _PACK, 2 * _PACK), lambda i: (0, 0)),
            pl.BlockSpec((1, 2 * _PACK), lambda i: (0, 0)),
        ],
        out_specs=[
            pl.BlockSpec((_BM, _PACK), lambda i: (i, 0)),
            pl.BlockSpec((_BM, _PACK), lambda i: (i, 0)),
        ],
        out_shape=[
            jax.ShapeDtypeStruct((nr, _PACK), X.dtype),
            jax.ShapeDtypeStruct((nr, _PACK), X.dtype),
        ],
        compiler_params=pltpu.CompilerParams(
            dimension_semantics=("parallel",),
        ),
    )(xr, w1bd, b1bd, w2bd, b2bd)
    return la.reshape(n), lb.reshape(n)
